# baseline (speedup 1.0000x reference)
import functools

import jax
import jax.numpy as jnp
from jax import lax
from jax.experimental import pallas as pl
from jax.experimental.pallas import tpu as pltpu
from jax.experimental.pallas import tpu_sc as plsc

MEM = 16384
DC = 512
DM = 1024
NTOK = 128
ROWS = 4096
RBLK = 256


def _sum_body(x_ref, o_ref):
    @pl.when(pl.program_id(0) == 0)
    def _():
        o_ref[...] = jnp.zeros_like(o_ref)

    o_ref[...] += jnp.sum(x_ref[...], axis=0, keepdims=True)


_sum_call = pl.pallas_call(
    _sum_body,
    grid=(ROWS // RBLK,),
    in_specs=[pl.BlockSpec((RBLK, DM), lambda i: (i, 0))],
    out_specs=pl.BlockSpec((1, DM), lambda i: (0, 0)),
    out_shape=jax.ShapeDtypeStruct((1, DM), jnp.float32),
)


def _select_body(i_ref, age_ref, out_ref):
    i_val = i_ref[0, 0]
    ridx = lax.broadcasted_iota(jnp.int32, (128, 128), 0)
    cidx = lax.broadcasted_iota(jnp.int32, (128, 128), 1)
    idxm = ridx * 128 + cidx
    idxf = idxm.astype(jnp.float32)
    a0 = jnp.where(idxm == i_val, 0.0, age_ref[...] + 1.0)
    lane = lax.broadcasted_iota(jnp.int32, (1, 128), 1)

    def body(p, carry):
        a, out = carry
        gmin = jnp.min(a)
        cand = jnp.where(a == gmin, idxf, 3.0e7)
        gidx = jnp.min(cand)
        out = out + gidx * (lane == p).astype(jnp.float32)
        a = jnp.where(idxf == gidx, jnp.inf, a)
        return a, out

    _, out = lax.fori_loop(
        0, NTOK, body, (a0, jnp.zeros((1, 128), jnp.float32))
    )
    out_ref[...] = out.astype(jnp.int32)


_select_call = pl.pallas_call(
    _select_body,
    in_specs=[
        pl.BlockSpec(memory_space=pltpu.SMEM),
        pl.BlockSpec((128, 128), lambda: (0, 0)),
    ],
    out_specs=pl.BlockSpec((1, 128), lambda: (0, 0)),
    out_shape=jax.ShapeDtypeStruct((1, 128), jnp.int32),
)

_GW = 16
_GR = NTOK // _GW


@functools.cache
def _make_gather():
    mesh = plsc.VectorSubcoreMesh(core_axis_name="c", subcore_axis_name="s")

    @functools.partial(
        pl.kernel,
        mesh=mesh,
        out_type=jax.ShapeDtypeStruct((NTOK, DC), jnp.float32),
        scratch_types=[
            pltpu.VMEM((_GR,), jnp.int32),
            pltpu.VMEM((_GR, DC), jnp.float32),
            pltpu.SemaphoreType.DMA,
        ],
    )
    def gather_k(vals_hbm, idx_hbm, out_hbm, idx_v, rows_v, sem):
        wid = lax.axis_index("s") * 2 + lax.axis_index("c")

        @pl.when(wid < _GW)
        def _():
            base = wid * _GR
            pltpu.sync_copy(idx_hbm.at[pl.ds(base, _GR)], idx_v)
            pltpu.async_copy(vals_hbm.at[idx_v], rows_v, sem).wait()
            pltpu.sync_copy(rows_v, out_hbm.at[pl.ds(base, _GR)])

    return gather_k


def _final_body(sum_ref, wval_ref, g_ref, wtok_ref, out_ref):
    v_new = (sum_ref[...] * (1.0 / ROWS)) @ wval_ref[...]
    rsel = lax.broadcasted_iota(jnp.int32, (NTOK, 1), 0) == 0
    rows = jnp.where(rsel, v_new, g_ref[...])
    out_ref[...] = rows @ wtok_ref[...]


_final_call = pl.pallas_call(
    _final_body,
    in_specs=[
        pl.BlockSpec((1, DM), lambda: (0, 0)),
        pl.BlockSpec((DM, DC), lambda: (0, 0)),
        pl.BlockSpec((NTOK, DC), lambda: (0, 0)),
        pl.BlockSpec((DC, DM), lambda: (0, 0)),
    ],
    out_specs=pl.BlockSpec((NTOK, DM), lambda: (0, 0)),
    out_shape=jax.ShapeDtypeStruct((NTOK, DM), jnp.float32),
)


def kernel(k_tok, v_tok, keys, vals, age, W_key, W_val, W_tok, ptr, n_tokens):
    B = k_tok.shape[0]
    sumv = _sum_call(v_tok.reshape(ROWS, DM))
    i = jnp.asarray(ptr % MEM, jnp.int32).reshape(1, 1)
    top = _select_call(i, age.reshape(128, 128))
    g = _make_gather()(vals, top.reshape(NTOK))
    out = _final_call(sumv, W_val, g, W_tok)
    return jnp.broadcast_to(out[None, :, :], (B, NTOK, DM))
